# Initial kernel scaffold; baseline (speedup 1.0000x reference)
import jax
import jax.numpy as jnp
from jax.experimental import pallas as pl


def kernel(features, labels, centers):
    raise NotImplementedError("write your pallas kernel here")


# speedup vs baseline: 2.6370x; 2.6370x over previous
import functools

import jax
import jax.numpy as jnp
from jax import lax
from jax.experimental import pallas as pl
from jax.experimental.pallas import tpu as pltpu
from jax.experimental.pallas import tpu_sc as plsc

NUM_CLASSES = 1000
FEAT = 64
BATCH = 16384
NC = 2
NS = 16
NW = NC * NS
BPW = BATCH // NW
GCH = 128
NG = BPW // GCH
CPAD = 1008


@functools.partial(
    pl.kernel,
    out_type=(
        jax.ShapeDtypeStruct((NW, CPAD), jnp.float32),
        jax.ShapeDtypeStruct((NW, CPAD), jnp.float32),
    ),
    mesh=plsc.VectorSubcoreMesh(core_axis_name="c", subcore_axis_name="s"),
    compiler_params=pltpu.CompilerParams(needs_layout_passes=False,
                                         use_tc_tiling_on_sc=False),
    scratch_types=[
        pltpu.VMEM((NG, GCH), jnp.int32),
        pltpu.VMEM((BPW, FEAT), jnp.float32),
        pltpu.VMEM((BPW, FEAT), jnp.float32),
        pltpu.VMEM((CPAD * 16,), jnp.float32),
        pltpu.VMEM((CPAD,), jnp.float32),
        pltpu.VMEM((CPAD,), jnp.float32),
        pltpu.SemaphoreType.DMA,
        pltpu.SemaphoreType.DMA,
    ],
)
def _sc_center_partials(feat_hbm, lab_hbm, cent_hbm, sums_out, cnt_out,
                        idx_v, feat_v, cent_v, sums_v, rsum_v, cnt_v,
                        sem_g, sem_f):
    wid = lax.axis_index("s") * NC + lax.axis_index("c")

    pltpu.sync_copy(lab_hbm.at[wid], idx_v)
    gathers = [
        pltpu.async_copy(cent_hbm.at[idx_v.at[g]],
                         cent_v.at[pl.ds(g * GCH, GCH)], sem_g)
        for g in range(NG)
    ]
    feat_cp = pltpu.async_copy(feat_hbm.at[wid], feat_v, sem_f)

    zeros16 = jnp.zeros((16,), jnp.float32)

    def _zero_sums(j, carry):
        for u in range(16):
            sums_v[pl.ds(j * 256 + u * 16, 16)] = zeros16
        return carry

    lax.fori_loop(0, CPAD // 16, _zero_sums, 0)

    for u in range(CPAD // 16):
        cnt_v[pl.ds(u * 16, 16)] = zeros16

    for cp in gathers:
        cp.wait()
    feat_cp.wait()

    iota16 = lax.iota(jnp.int32, 16)
    ones16 = jnp.ones((16,), jnp.float32)

    for g in range(NG):
        def _body(kk, carry, g=g):
            base = g * GCH + kk * 16
            lab16 = idx_v[g, pl.ds(kk * 16, 16)]
            for lane in range(16):
                i = base + lane
                acc = zeros16
                for c in range(FEAT // 16):
                    d = (feat_v[i, pl.ds(c * 16, 16)]
                         - cent_v[i, pl.ds(c * 16, 16)])
                    acc = acc + d * d
                row_idx = lab16[lane] * 16 + iota16
                plsc.addupdate_scatter(sums_v, [row_idx], acc)
                plsc.addupdate_scatter(cnt_v, [lab16], ones16,
                                       mask=iota16 == lane)
            return carry

        lax.fori_loop(0, GCH // 16, _body, 0)

    iota_x16 = iota16 * 16

    def _rowsum(j, carry):
        col0 = j * 256 + iota_x16
        tot = zeros16
        for c in range(16):
            tot = tot + plsc.load_gather(sums_v, [col0 + c])
        rsum_v[pl.ds(j * 16, 16)] = tot
        return carry

    lax.fori_loop(0, CPAD // 16, _rowsum, 0)

    pltpu.sync_copy(rsum_v, sums_out.at[wid])
    pltpu.sync_copy(cnt_v, cnt_out.at[wid])


def _finish_body(sums_ref, cnt_ref, out_ref):
    s = jnp.sum(sums_ref[...], axis=0)
    n = jnp.sum(cnt_ref[...], axis=0)
    denom = jnp.maximum(n, 1.0) * FEAT
    per_class = jnp.where(n > 0, s / denom, 0.0)
    out_ref[...] = (jnp.sum(per_class) / BATCH).reshape(1, 1)


def kernel(features, labels, centers):
    feat_r = features.reshape(NW, BPW, FEAT)
    lab_r = labels.astype(jnp.int32).reshape(NW, NG, GCH)
    part_sums, part_cnt = _sc_center_partials(feat_r, lab_r, centers)
    loss = pl.pallas_call(
        _finish_body,
        out_shape=jax.ShapeDtypeStruct((1, 1), jnp.float32),
    )(part_sums, part_cnt)
    return loss.reshape(())


# speedup vs baseline: 2.8142x; 1.0672x over previous
import functools

import jax
import jax.numpy as jnp
from jax import lax
from jax.experimental import pallas as pl
from jax.experimental.pallas import tpu as pltpu
from jax.experimental.pallas import tpu_sc as plsc

NUM_CLASSES = 1000
FEAT = 64
BATCH = 16384
NC = 2
NS = 16
NW = NC * NS
BPW = BATCH // NW
GCH = 128
NG = BPW // GCH
CPAD = 1008


@functools.partial(
    pl.kernel,
    out_type=(
        jax.ShapeDtypeStruct((NW, CPAD), jnp.float32),
        jax.ShapeDtypeStruct((NW, CPAD), jnp.float32),
    ),
    mesh=plsc.VectorSubcoreMesh(core_axis_name="c", subcore_axis_name="s"),
    compiler_params=pltpu.CompilerParams(needs_layout_passes=False,
                                         use_tc_tiling_on_sc=False),
    scratch_types=[
        pltpu.VMEM((NG, GCH), jnp.int32),
        pltpu.VMEM((BPW, FEAT), jnp.float32),
        pltpu.VMEM((BPW, FEAT), jnp.float32),
        pltpu.VMEM((CPAD * 16,), jnp.float32),
        pltpu.VMEM((CPAD,), jnp.float32),
        pltpu.VMEM((CPAD,), jnp.float32),
        pltpu.SemaphoreType.DMA,
        pltpu.SemaphoreType.DMA,
    ],
)
def _sc_center_partials(feat_hbm, lab_hbm, cent_hbm, sums_out, cnt_out,
                        idx_v, feat_v, cent_v, sums_v, rsum_v, cnt_v,
                        sem_g, sem_f):
    wid = lax.axis_index("s") * NC + lax.axis_index("c")

    pltpu.sync_copy(lab_hbm.at[wid], idx_v)
    gathers = [
        pltpu.async_copy(cent_hbm.at[idx_v.at[g]],
                         cent_v.at[pl.ds(g * GCH, GCH)], sem_g)
        for g in range(NG)
    ]
    feat_cp = pltpu.async_copy(feat_hbm.at[wid], feat_v, sem_f)

    zeros16 = jnp.zeros((16,), jnp.float32)

    @plsc.parallel_loop(0, CPAD // 16, unroll=2)
    def _zero_sums(j):
        for u in range(16):
            sums_v[pl.ds(j * 256 + u * 16, 16)] = zeros16

    for u in range(CPAD // 16):
        cnt_v[pl.ds(u * 16, 16)] = zeros16

    for cp in gathers:
        cp.wait()
    feat_cp.wait()

    iota16 = lax.iota(jnp.int32, 16)
    ones16 = jnp.ones((16,), jnp.float32)

    @plsc.parallel_loop(0, BPW // 16, unroll=2)
    def _accumulate(kk):
        base = kk * 16
        lab16 = idx_v[kk // 8, pl.ds((kk % 8) * 16, 16)]
        for lane in range(16):
            i = base + lane
            acc = zeros16
            for c in range(FEAT // 16):
                d = (feat_v[i, pl.ds(c * 16, 16)]
                     - cent_v[i, pl.ds(c * 16, 16)])
                acc = acc + d * d
            row_idx = lab16[lane] * 16 + iota16
            plsc.addupdate_scatter(sums_v, [row_idx], acc)
            plsc.addupdate_scatter(cnt_v, [lab16], ones16,
                                   mask=iota16 == lane)

    iota_x16 = iota16 * 16

    @plsc.parallel_loop(0, CPAD // 16, unroll=2)
    def _rowsum(j):
        col0 = j * 256 + iota_x16
        tot = zeros16
        for c in range(16):
            tot = tot + plsc.load_gather(sums_v, [col0 + c])
        rsum_v[pl.ds(j * 16, 16)] = tot

    pltpu.sync_copy(rsum_v, sums_out.at[wid])
    pltpu.sync_copy(cnt_v, cnt_out.at[wid])


def _finish_body(sums_ref, cnt_ref, out_ref):
    s = jnp.sum(sums_ref[...], axis=0)
    n = jnp.sum(cnt_ref[...], axis=0)
    denom = jnp.maximum(n, 1.0) * FEAT
    per_class = jnp.where(n > 0, s / denom, 0.0)
    out_ref[...] = (jnp.sum(per_class) / BATCH).reshape(1, 1)


def kernel(features, labels, centers):
    feat_r = features.reshape(NW, BPW, FEAT)
    lab_r = labels.astype(jnp.int32).reshape(NW, NG, GCH)
    part_sums, part_cnt = _sc_center_partials(feat_r, lab_r, centers)
    loss = pl.pallas_call(
        _finish_body,
        out_shape=jax.ShapeDtypeStruct((1, 1), jnp.float32),
    )(part_sums, part_cnt)
    return loss.reshape(())
